# Initial kernel scaffold; baseline (speedup 1.0000x reference)
import jax
import jax.numpy as jnp
from jax.experimental import pallas as pl


def kernel(x, Wit, bit, Wft, bft, W1, b1, W2, b2, Wnu, bnu, Wot, bot, Wp, bp):
    raise NotImplementedError("write your pallas kernel here")


# speedup vs baseline: 3.2625x; 3.2625x over previous
import functools
import math

import numpy as np
import jax
import jax.numpy as jnp
from jax import lax
from jax.experimental import pallas as pl
from jax.experimental.pallas import tpu as pltpu
from jax.experimental.pallas import tpu_sc as plsc

B = 8
N = 1024
D = 256
K = 8
OUT = 512
DILATIONS = [1, 2, 4, 8]

_SQRT_C = np.sqrt(np.float32(1.0 + 1e-5))
_HALF_SQRT2 = np.float32(0.707106769)


def _bn(x):
    return x / _SQRT_C


def _erfc(u):
    a = jnp.abs(u)
    x2 = u * u
    pe = jnp.float32(7.85386146e-05)
    for c in (-0.000801019371, 0.00518832775, -0.0268538129, 0.112835854,
              -0.37612626, 1.12837911):
        pe = pe * x2 + jnp.float32(c)
    res_lt1 = 1.0 - u * pe
    nx2 = -x2
    e = jnp.exp(nx2)
    q = e * (1.0 / a)
    z = 1.0 / x2
    pa = jnp.float32(0.0232682)
    for c in (-0.138703942, 0.368742466, -0.582473278, 0.621000469,
              -0.494451523, 0.340488, -0.274112701, 0.563825965):
        pa = pa * z + jnp.float32(c)
    pb = z * jnp.float32(-10.477664) + jnp.float32(12.9772)
    for c in (-7.49551868, 2.92101908, -1.01526523, 0.42184633,
              -0.282076746, 0.564189494):
        pb = pb * z + jnp.float32(c)
    p = jnp.where(a < 2.0, pa, pb)
    val = q * p
    val = jnp.where(nx2 < jnp.float32(-88.7228394), 0.0, val)
    res_ge1 = jnp.where(u < 0.0, 2.0 - val, val)
    return jnp.where(a < 1.0, res_lt1, res_ge1)


def _gelu(x):
    return (x * 0.5) * _erfc((-x) * _HALF_SQRT2)


def _b16(v):
    return v.astype(jnp.bfloat16)


def _dot1x(a, b):
    return jnp.dot(_b16(a), _b16(b), preferred_element_type=jnp.float32)


def _block_a_body(d, x_ref, Wit_ref, bit_ref, Wft_ref, bft_ref,
                  xt_ref, idx_ref, pd_ref):
    b = pl.program_id(0)
    x = x_ref[0]
    xt = _gelu(_bn(_dot1x(x, Wit_ref[...]) + bit_ref[...]))
    xt_ref[0] = xt

    xf = _dot1x(xt, Wft_ref[...]) + bft_ref[...]
    sq = xf * xf
    xx = jnp.sum(sq, axis=1, keepdims=True)
    xxr = lax.transpose(xx, (1, 0))
    xfb = _b16(xf)
    xy = lax.dot_general(xfb, xfb, (((1,), (1,)), ((), ())),
                         preferred_element_type=jnp.float32)
    pd_ref[...] = xx + xxr - 2.0 * xy

    iota8 = lax.broadcasted_iota(jnp.int32, (N, K), 1)
    num_ranks = 2 + 7 * d

    def step(r, idx_acc):
        pdv = pd_ref[...]
        m = jnp.min(pdv, axis=1, keepdims=True)
        iota_col = lax.broadcasted_iota(jnp.int32, (N, N), 1)
        am = jnp.min(jnp.where(pdv == m, iota_col, N), axis=1,
                     keepdims=True)
        pd_ref[...] = jnp.where(iota_col == am, jnp.inf, pdv)
        sel = (r >= 1) & (((r - 1) % d) == 0)
        slot = jnp.where(sel, (r - 1) // d, -1)
        return jnp.where(iota8 == slot, jnp.broadcast_to(am, (N, K)),
                         idx_acc)

    idx = lax.fori_loop(0, num_ranks, step, jnp.zeros((N, K), jnp.int32))
    idx_ref[0] = idx + b * N


def _make_block_a(d):
    full = lambda shape: pl.BlockSpec(shape, lambda b: tuple(0 for _ in shape))
    return pl.pallas_call(
        functools.partial(_block_a_body, d),
        grid=(B,),
        in_specs=[
            pl.BlockSpec((1, N, D), lambda b: (b, 0, 0)),
            full((D, D)), full((1, D)), full((D, D)), full((1, D)),
        ],
        out_specs=[
            pl.BlockSpec((1, N, D), lambda b: (b, 0, 0)),
            pl.BlockSpec((1, N, K), lambda b: (b, 0, 0)),
        ],
        out_shape=[
            jax.ShapeDtypeStruct((B, N, D), jnp.float32),
            jax.ShapeDtypeStruct((B, N, K), jnp.int32),
        ],
        scratch_shapes=[pltpu.VMEM((N, N), jnp.float32)],
    )


_IDX_ROWS = B * N * K // 128


def _sc_gather(table, idx2d):
    info = plsc.get_sparse_core_info()
    nw = info.num_cores * info.num_subcores
    rows_per_w = _IDX_ROWS // nw
    mesh = plsc.VectorSubcoreMesh(core_axis_name="c", subcore_axis_name="s")

    @functools.partial(
        pl.kernel, mesh=mesh,
        out_type=jax.ShapeDtypeStruct((B * N * K, D), jnp.float32),
        scratch_types=[
            pltpu.VMEM((rows_per_w, 128), jnp.int32),
            pltpu.VMEM((128, D), jnp.float32),
            pltpu.VMEM((128, D), jnp.float32),
            pltpu.SemaphoreType.DMA,
            pltpu.SemaphoreType.DMA,
        ],
    )
    def k(table_hbm, idx_hbm, out_hbm, idx_v, buf0, buf1, sem0, sem1):
        wid = lax.axis_index("s") * info.num_cores + lax.axis_index("c")
        base = wid * rows_per_w
        pltpu.sync_copy(idx_hbm.at[pl.ds(base, rows_per_w), :], idx_v)
        bufs = (buf0, buf1)
        sems = (sem0, sem1)
        handles = [None, None]
        for j in range(rows_per_w):
            handles[j % 2] = pltpu.async_copy(
                table_hbm.at[idx_v.at[j]], bufs[j % 2], sems[j % 2])
            if j > 0:
                handles[(j - 1) % 2].wait()
                pltpu.sync_copy(
                    bufs[(j - 1) % 2],
                    out_hbm.at[pl.ds((base + j - 1) * 128, 128), :])
        handles[(rows_per_w - 1) % 2].wait()
        pltpu.sync_copy(
            bufs[(rows_per_w - 1) % 2],
            out_hbm.at[pl.ds((base + rows_per_w - 1) * 128, 128), :])

    return k(table, idx2d)


def _block_b_body(x_ref, xt_ref, E_ref, W1_ref, b1_ref, W2p_ref, b2_ref,
                  Wnu_ref, bnu_ref, Wot_ref, bot_ref, out_ref):
    xt = xt_ref[0]
    E3 = E_ref[0].reshape(N, K, D)
    edge = E3 - xt[:, None, :]
    ef = edge.reshape(N * K, D)
    hh = _gelu(_bn(_dot1x(ef, W1_ref[...]) + b1_ref[...]))
    hw = _dot1x(hh, W2p_ref[...])[:, 0:1] + b2_ref[0, 0]
    w = 1.0 / (1.0 + jnp.exp(-hw))
    weighted = (ef * w).reshape(N, K, D)
    agg = jnp.sum(weighted, axis=1)
    comb = jnp.concatenate([xt, agg], axis=1)
    upd = _gelu(_bn(_dot1x(comb, Wnu_ref[...]) + bnu_ref[...]))
    o = _gelu(_bn(_dot1x(upd, Wot_ref[...]) + bot_ref[...]))
    out_ref[0] = o + x_ref[0]


def _make_block_b():
    full = lambda shape: pl.BlockSpec(shape, lambda b: tuple(0 for _ in shape))
    return pl.pallas_call(
        _block_b_body,
        grid=(B,),
        in_specs=[
            pl.BlockSpec((1, N, D), lambda b: (b, 0, 0)),
            pl.BlockSpec((1, N, D), lambda b: (b, 0, 0)),
            pl.BlockSpec((1, N * K, D), lambda b: (b, 0, 0)),
            full((D, D // 2)), full((1, D // 2)),
            full((D // 2, D // 2)), full((1, 1)),
            full((2 * D, D)), full((1, D)),
            full((D, D)), full((1, D)),
        ],
        out_specs=pl.BlockSpec((1, N, D), lambda b: (b, 0, 0)),
        out_shape=jax.ShapeDtypeStruct((B, N, D), jnp.float32),
    )


def _final_body(h_ref, Wp_ref, bp_ref, out_ref):
    feats = []
    for b in range(B):
        feats.append(jnp.sum(h_ref[b], axis=0, keepdims=True) * (1.0 / N))
    feat = jnp.concatenate(feats, axis=0)
    out_ref[...] = _gelu(_bn(_dot1x(feat, Wp_ref[...]) + bp_ref[...]))


_final_call = pl.pallas_call(
    _final_body,
    out_shape=jax.ShapeDtypeStruct((B, OUT), jnp.float32),
)

_block_a_calls = [_make_block_a(d) for d in DILATIONS]
_block_b_call = _make_block_b()


def kernel(x, Wit, bit, Wft, bft, W1, b1, W2, b2, Wnu, bnu, Wot, bot, Wp, bp):
    h = x
    for i in range(len(DILATIONS)):
        xt, idxf = _block_a_calls[i](
            h, Wit[i], bit[i].reshape(1, D), Wft[i], bft[i].reshape(1, D))
        E = _sc_gather(xt.reshape(B * N, D),
                       idxf.reshape(_IDX_ROWS, 128))
        W2p = jnp.pad(W2[i], ((0, 0), (0, D // 2 - 1)))
        h = _block_b_call(
            h, xt, E.reshape(B, N * K, D),
            W1[i], b1[i].reshape(1, D // 2), W2p, b2[i].reshape(1, 1),
            Wnu[i], bnu[i].reshape(1, D),
            Wot[i], bot[i].reshape(1, D))
    return _final_call(h, Wp, bp.reshape(1, OUT))
